# Initial kernel scaffold; baseline (speedup 1.0000x reference)
import functools

import jax
import jax.numpy as jnp
from jax import lax
from jax.experimental import pallas as pl
from jax.experimental.pallas import tpu as pltpu
from jax.experimental.pallas import tpu_sc as plsc

_B, _T = 16384, 200
_V, _D = 90, 64
_N = _B * _T
_NC, _NS = 2, 16
_NW = _NC * _NS
_PER_W = _N // _NW
_GW = 128
_G = 8
_C = _G * _GW
_ITERS = _PER_W // _C


def _sc_body(idx_hbm, table_hbm, out_hbm, table_v, idx_v, rows_v, sem):
    wid = lax.axis_index("s") * _NC + lax.axis_index("c")
    pltpu.sync_copy(table_hbm, table_v)
    base_row = wid * (_PER_W // _GW)
    base = wid * _PER_W

    def chunk(i, carry):
        pltpu.sync_copy(idx_hbm.at[pl.ds(base_row + i * _G, _G)], idx_v)
        copies = []
        for j in range(_G):
            copies.append(
                pltpu.async_copy(
                    table_v.at[idx_v.at[j]],
                    rows_v.at[pl.ds(j * _GW, _GW)],
                    sem,
                )
            )
        for c in copies:
            c.wait()
        pltpu.sync_copy(rows_v, out_hbm.at[pl.ds(base + i * _C, _C)])
        return carry

    lax.fori_loop(0, _ITERS, chunk, 0)


@jax.jit
def _lookup(idx2d, table):
    mesh = plsc.VectorSubcoreMesh(core_axis_name="c", subcore_axis_name="s")
    run = functools.partial(
        pl.kernel,
        out_type=jax.ShapeDtypeStruct((_N, _D), jnp.float32),
        mesh=mesh,
        scratch_types=[
            pltpu.VMEM((_V, _D), jnp.float32),
            pltpu.VMEM((_G, _GW), jnp.int32),
            pltpu.VMEM((_C, _D), jnp.float32),
            pltpu.SemaphoreType.DMA,
        ],
    )(_sc_body)
    return run(idx2d, table)


def kernel(indices, table):
    idx2d = indices.astype(jnp.int32).reshape(_N // _GW, _GW)
    out = _lookup(idx2d, table)
    return out.reshape(_B, _T, _D)


# speedup vs baseline: 2.9446x; 2.9446x over previous
import functools

import jax
import jax.numpy as jnp
from jax import lax
from jax.experimental import pallas as pl
from jax.experimental.pallas import tpu as pltpu
from jax.experimental.pallas import tpu_sc as plsc

_B, _T = 16384, 200
_V, _D = 90, 64
_N = _B * _T
_NC, _NS = 2, 16
_NW = _NC * _NS
_PER_W = _N // _NW
_GW = 128
_G = 8
_C = _G * _GW
_ITERS = _PER_W // _C


def _sc_body(idx_hbm, table_hbm, out_hbm, idx_v, rows_v, sem):
    wid = lax.axis_index("s") * _NC + lax.axis_index("c")
    base_row = wid * (_PER_W // _GW)
    base = wid * _PER_W

    def chunk(i, carry):
        pltpu.sync_copy(idx_hbm.at[pl.ds(base_row + i * _G, _G)], idx_v)
        copies = []
        for j in range(_G):
            copies.append(
                pltpu.async_copy(
                    table_hbm.at[idx_v.at[j]],
                    rows_v.at[pl.ds(j * _GW, _GW)],
                    sem,
                )
            )
        for c in copies:
            c.wait()
        pltpu.sync_copy(rows_v, out_hbm.at[pl.ds(base + i * _C, _C)])
        return carry

    lax.fori_loop(0, _ITERS, chunk, 0)


@jax.jit
def _lookup(idx2d, table):
    mesh = plsc.VectorSubcoreMesh(core_axis_name="c", subcore_axis_name="s")
    run = functools.partial(
        pl.kernel,
        out_type=jax.ShapeDtypeStruct((_N, _D), jnp.float32),
        mesh=mesh,
        scratch_types=[
            pltpu.VMEM((_G, _GW), jnp.int32),
            pltpu.VMEM((_C, _D), jnp.float32),
            pltpu.SemaphoreType.DMA,
        ],
        compiler_params=pltpu.CompilerParams(use_tc_tiling_on_sc=False),
    )(_sc_body)
    return run(idx2d, table)


def kernel(indices, table):
    idx2d = indices.astype(jnp.int32).reshape(_N // _GW, _GW)
    out = _lookup(idx2d, table)
    return out.reshape(_B, _T, _D)


# speedup vs baseline: 5.6561x; 1.9208x over previous
import functools

import jax
import jax.numpy as jnp
from jax import lax
from jax.experimental import pallas as pl
from jax.experimental.pallas import tpu as pltpu
from jax.experimental.pallas import tpu_sc as plsc

_B, _T = 16384, 200
_V, _D = 90, 64
_N = _B * _T
_NC, _NS = 2, 16
_NW = _NC * _NS
_PER_W = _N // _NW
_GW = 128
_G = 5
_C = _G * _GW
_ITERS = _PER_W // _C
_NB = 2


def _sc_body(idx_hbm, table_hbm, out_hbm, table_s, idx_v, rows_v, gsem, osem):
    sid = lax.axis_index("s")
    wid = sid * _NC + lax.axis_index("c")
    @pl.when(sid == 0)
    def _stage():
        pltpu.sync_copy(table_hbm, table_s)

    plsc.subcore_barrier()
    base_row = wid * (_PER_W // _GW)
    base = wid * _PER_W

    def super_chunk(k, carry):
        for b in range(_NB):
            i = k * _NB + b
            @pl.when(k > 0)
            def _reclaim():
                pltpu.make_async_copy(
                    rows_v.at[b], out_hbm.at[pl.ds(0, _C)], osem
                ).wait()

            pltpu.sync_copy(idx_hbm.at[pl.ds(base_row + i * _G, _G)], idx_v.at[b])
            copies = []
            for j in range(_G):
                copies.append(
                    pltpu.async_copy(
                        table_s.at[idx_v.at[b, j]],
                        rows_v.at[b, pl.ds(j * _GW, _GW)],
                        gsem,
                    )
                )
            for c in copies:
                c.wait()
            pltpu.async_copy(rows_v.at[b], out_hbm.at[pl.ds(base + i * _C, _C)], osem)
        return carry

    lax.fori_loop(0, _ITERS // _NB, super_chunk, 0)
    for b in range(_NB):
        pltpu.make_async_copy(rows_v.at[b], out_hbm.at[pl.ds(0, _C)], osem).wait()


@jax.jit
def _lookup(idx2d, table):
    mesh = plsc.VectorSubcoreMesh(core_axis_name="c", subcore_axis_name="s")
    run = functools.partial(
        pl.kernel,
        out_type=jax.ShapeDtypeStruct((_N, _D), jnp.float32),
        mesh=mesh,
        scratch_types=[
            pltpu.VMEM_SHARED((_V, _D), jnp.float32),
            pltpu.VMEM((_NB, _G, _GW), jnp.int32),
            pltpu.VMEM((_NB, _C, _D), jnp.float32),
            pltpu.SemaphoreType.DMA,
            pltpu.SemaphoreType.DMA,
        ],
        compiler_params=pltpu.CompilerParams(use_tc_tiling_on_sc=False),
    )(_sc_body)
    return run(idx2d, table)


def kernel(indices, table):
    idx2d = indices.astype(jnp.int32).reshape(_N // _GW, _GW)
    out = _lookup(idx2d, table)
    return out.reshape(_B, _T, _D)
